# baseline (speedup 1.0000x reference)
import functools

import jax
import jax.numpy as jnp
from jax import lax
from jax.experimental import pallas as pl
from jax.experimental.pallas import tpu as pltpu

TOPK = 5
FEA_GATE_TH = 0.15
ROW_BLOCK = 8
NEG_INF = float("-inf")
BIGP = 2**30
LANE = 128
STEP = 2 * LANE


def _insert(ts, ps, c, q):
    for j in range(TOPK):
        cond = c > ts[j]
        nt = jnp.where(cond, c, ts[j])
        nc = jnp.where(cond, ts[j], c)
        np_ = jnp.where(cond, q, ps[j])
        nq = jnp.where(cond, ps[j], q)
        ts[j], ps[j], c, q = nt, np_, nc, nq
    return ts, ps


def _decode_body(logits_ref, counts_ref, gates_ref, noise_ref,
                 word_ref, prob_ref, out_counts_ref):
    rb, v = logits_ref.shape
    ns = 2
    step = ns * LANE
    nfull = (v - LANE) // step
    base_tail = nfull * step
    lane_iota = lax.broadcasted_iota(jnp.int32, (rb, LANE), 1)
    lane4 = lane_iota << 2

    def slice_pair(base):
        c = logits_ref[:, pl.ds(base, LANE)]
        q = counts_ref[:, pl.ds(base, LANE)] + (lane4 + (base << 2))
        return c, q

    init = []
    for _ in range(ns):
        init += [jnp.full((rb, LANE), NEG_INF, jnp.float32)
                 for _ in range(TOPK)]
        init += [jnp.full((rb, LANE), BIGP, jnp.int32) for _ in range(TOPK)]

    def body(s, carry):
        carry = list(carry)
        base = pl.multiple_of(s * step, step)
        for k in range(ns):
            o = 2 * TOPK * k
            ts, ps = carry[o:o + TOPK], carry[o + TOPK:o + 2 * TOPK]
            c, q = slice_pair(base + k * LANE)
            ts, ps = _insert(ts, ps, c, q)
            carry[o:o + TOPK] = ts
            carry[o + TOPK:o + 2 * TOPK] = ps
        return tuple(carry)

    carry = list(lax.fori_loop(0, nfull, body, tuple(init)))
    nfull_tail = (v - base_tail) // LANE
    for k in range(nfull_tail):
        o = 2 * TOPK * k
        ts, ps = carry[o:o + TOPK], carry[o + TOPK:o + 2 * TOPK]
        c, q = slice_pair(base_tail + k * LANE)
        ts, ps = _insert(ts, ps, c, q)
        carry[o:o + TOPK] = ts
        carry[o + TOPK:o + 2 * TOPK] = ps
    novl = LANE - (v - base_tail - nfull_tail * LANE)
    keep = lane_iota >= novl
    if novl < LANE:
        c = logits_ref[:, pl.ds(v - LANE, LANE)]
        q = counts_ref[:, pl.ds(v - LANE, LANE)] + (lane4 + ((v - LANE) << 2))
        c = jnp.where(keep, c, NEG_INF)
        q = jnp.where(keep, q, BIGP)
        o = 2 * TOPK * (ns - 1)
        ts, ps = carry[o:o + TOPK], carry[o + TOPK:o + 2 * TOPK]
        ts, ps = _insert(ts, ps, c, q)
        carry[o:o + TOPK] = ts
        carry[o + TOPK:o + 2 * TOPK] = ps

    t_all = jnp.concatenate(
        [t for k in range(ns) for t in carry[2 * TOPK * k:2 * TOPK * k + TOPK]],
        axis=-1)
    p_all = jnp.concatenate(
        [p for k in range(ns)
         for p in carry[2 * TOPK * k + TOPK:2 * TOPK * (k + 1)]], axis=-1)
    vals, pks = [], []
    for _ in range(TOPK):
        vi = jnp.max(t_all, axis=-1, keepdims=True)
        pi = jnp.min(jnp.where(t_all == vi, p_all, BIGP), axis=-1,
                     keepdims=True)
        t_all = jnp.where((t_all == vi) & (p_all == pi), NEG_INF, t_all)
        vals.append(vi)
        pks.append(pi)
    idxs = [p >> 2 for p in pks]
    cnts = [p & 3 for p in pks]

    m0 = vals[0]
    s_sum = jnp.sum(jnp.exp(logits_ref[...] - m0), axis=-1, keepdims=True)

    qs = [jnp.exp(vi - m0) / s_sum for vi in vals]

    gate = gates_ref[...] > FEA_GATE_TH
    pprobs = [jnp.where(gate, qi / (1.0 + 2.0 * ci.astype(jnp.float32)), qi)
              for qi, ci in zip(qs, cnts)]

    total = pprobs[0] + pprobs[1] + pprobs[2] + pprobs[3] + pprobs[4]
    u = noise_ref[...] * total
    cdf = pprobs[0]
    choice = (cdf < u).astype(jnp.int32)
    for i in range(1, TOPK):
        cdf = cdf + pprobs[i]
        choice = choice + (cdf < u).astype(jnp.int32)
    choice = jnp.clip(choice, 0, TOPK - 1)

    word = idxs[TOPK - 1]
    prob = qs[TOPK - 1]
    for i in range(TOPK - 2, -1, -1):
        pick = choice == i
        word = jnp.where(pick, idxs[i], word)
        prob = jnp.where(pick, qs[i], prob)

    word_ref[...] = word
    prob_ref[...] = prob

    ckey = jnp.where(gate, word, -1)
    full_iota = lax.broadcasted_iota(jnp.int32, (rb, v), 1)
    out_counts_ref[...] = (counts_ref[...]
                           + (full_iota == ckey).astype(jnp.int32))


@functools.partial(jax.jit, static_argnames=())
def kernel(logits, feature_counts, fea_gates, noise):
    b, v = logits.shape
    rb = ROW_BLOCK
    grid = (b // rb,)
    row_spec = pl.BlockSpec((rb, v), lambda i: (i, 0))
    col_spec = pl.BlockSpec((rb, 1), lambda i: (i, 0))

    word, prob, new_counts = pl.pallas_call(
        _decode_body,
        grid=grid,
        compiler_params=pltpu.CompilerParams(
            dimension_semantics=("parallel",)),
        in_specs=[row_spec, row_spec, col_spec, col_spec],
        out_specs=[col_spec, col_spec, row_spec],
        out_shape=[
            jax.ShapeDtypeStruct((b, 1), jnp.int32),
            jax.ShapeDtypeStruct((b, 1), jnp.float32),
            jax.ShapeDtypeStruct((b, v), jnp.int32),
        ],
    )(logits, feature_counts, fea_gates.reshape(b, 1), noise.reshape(b, 1))
    return word.reshape(b), prob.reshape(b), new_counts


# speedup vs baseline: 1.0968x; 1.0968x over previous
import functools

import jax
import jax.numpy as jnp
from jax import lax
from jax.experimental import pallas as pl
from jax.experimental.pallas import tpu as pltpu

TOPK = 5
FEA_GATE_TH = 0.15
ROW_BLOCK = 8
NEG_INF = float("-inf")
BIGP = 2**30
LANE = 128
STEP = 2 * LANE


def _insert(ts, ps, c, q):
    for j in range(TOPK):
        cond = c > ts[j]
        nt = jnp.where(cond, c, ts[j])
        nc = jnp.where(cond, ts[j], c)
        np_ = jnp.where(cond, q, ps[j])
        nq = jnp.where(cond, ps[j], q)
        ts[j], ps[j], c, q = nt, np_, nc, nq
    return ts, ps


def _decode_body(logits_ref, counts_ref, gates_ref, noise_ref,
                 word_ref, prob_ref, out_counts_ref):
    rb, v = logits_ref.shape
    ns = 4
    unroll = 2
    step = ns * unroll * LANE
    nfull = (v - LANE) // step
    base_tail = nfull * step
    lane_iota = lax.broadcasted_iota(jnp.int32, (rb, LANE), 1)
    lane4 = lane_iota << 2

    def slice_pair(base):
        c = logits_ref[:, pl.ds(base, LANE)]
        q = counts_ref[:, pl.ds(base, LANE)] + (lane4 + (base << 2))
        return c, q

    init = []
    for _ in range(ns):
        init += [jnp.full((rb, LANE), NEG_INF, jnp.float32)
                 for _ in range(TOPK)]
        init += [jnp.full((rb, LANE), BIGP, jnp.int32) for _ in range(TOPK)]

    def body(s, carry):
        carry = list(carry)
        base = pl.multiple_of(s * step, step)
        for k in range(ns * unroll):
            o = 2 * TOPK * (k % ns)
            ts, ps = carry[o:o + TOPK], carry[o + TOPK:o + 2 * TOPK]
            c, q = slice_pair(base + k * LANE)
            ts, ps = _insert(ts, ps, c, q)
            carry[o:o + TOPK] = ts
            carry[o + TOPK:o + 2 * TOPK] = ps
        return tuple(carry)

    carry = list(lax.fori_loop(0, nfull, body, tuple(init)))
    nfull_tail = (v - base_tail) // LANE
    for k in range(nfull_tail):
        o = 2 * TOPK * (k % ns)
        ts, ps = carry[o:o + TOPK], carry[o + TOPK:o + 2 * TOPK]
        c, q = slice_pair(base_tail + k * LANE)
        ts, ps = _insert(ts, ps, c, q)
        carry[o:o + TOPK] = ts
        carry[o + TOPK:o + 2 * TOPK] = ps
    novl = LANE - (v - base_tail - nfull_tail * LANE)
    keep = lane_iota >= novl
    if novl < LANE:
        c = logits_ref[:, pl.ds(v - LANE, LANE)]
        q = counts_ref[:, pl.ds(v - LANE, LANE)] + (lane4 + ((v - LANE) << 2))
        c = jnp.where(keep, c, NEG_INF)
        q = jnp.where(keep, q, BIGP)
        o = 2 * TOPK * (ns - 1)
        ts, ps = carry[o:o + TOPK], carry[o + TOPK:o + 2 * TOPK]
        ts, ps = _insert(ts, ps, c, q)
        carry[o:o + TOPK] = ts
        carry[o + TOPK:o + 2 * TOPK] = ps

    t_all = jnp.concatenate(
        [t for k in range(ns) for t in carry[2 * TOPK * k:2 * TOPK * k + TOPK]],
        axis=-1)
    p_all = jnp.concatenate(
        [p for k in range(ns)
         for p in carry[2 * TOPK * k + TOPK:2 * TOPK * (k + 1)]], axis=-1)
    vals, pks = [], []
    for _ in range(TOPK):
        vi = jnp.max(t_all, axis=-1, keepdims=True)
        pi = jnp.min(jnp.where(t_all == vi, p_all, BIGP), axis=-1,
                     keepdims=True)
        t_all = jnp.where((t_all == vi) & (p_all == pi), NEG_INF, t_all)
        vals.append(vi)
        pks.append(pi)
    idxs = [p >> 2 for p in pks]
    cnts = [p & 3 for p in pks]

    m0 = vals[0]
    s_sum = jnp.sum(jnp.exp(logits_ref[...] - m0), axis=-1, keepdims=True)

    qs = [jnp.exp(vi - m0) / s_sum for vi in vals]

    gate = gates_ref[...] > FEA_GATE_TH
    pprobs = [jnp.where(gate, qi / (1.0 + 2.0 * ci.astype(jnp.float32)), qi)
              for qi, ci in zip(qs, cnts)]

    total = pprobs[0] + pprobs[1] + pprobs[2] + pprobs[3] + pprobs[4]
    u = noise_ref[...] * total
    cdf = pprobs[0]
    choice = (cdf < u).astype(jnp.int32)
    for i in range(1, TOPK):
        cdf = cdf + pprobs[i]
        choice = choice + (cdf < u).astype(jnp.int32)
    choice = jnp.clip(choice, 0, TOPK - 1)

    word = idxs[TOPK - 1]
    prob = qs[TOPK - 1]
    for i in range(TOPK - 2, -1, -1):
        pick = choice == i
        word = jnp.where(pick, idxs[i], word)
        prob = jnp.where(pick, qs[i], prob)

    word_ref[...] = word
    prob_ref[...] = prob

    ckey = jnp.where(gate, word, -1)
    full_iota = lax.broadcasted_iota(jnp.int32, (rb, v), 1)
    out_counts_ref[...] = (counts_ref[...]
                           + (full_iota == ckey).astype(jnp.int32))


@functools.partial(jax.jit, static_argnames=())
def kernel(logits, feature_counts, fea_gates, noise):
    b, v = logits.shape
    rb = ROW_BLOCK
    grid = (b // rb,)
    row_spec = pl.BlockSpec((rb, v), lambda i: (i, 0))
    col_spec = pl.BlockSpec((rb, 1), lambda i: (i, 0))

    word, prob, new_counts = pl.pallas_call(
        _decode_body,
        grid=grid,
        compiler_params=pltpu.CompilerParams(
            dimension_semantics=("parallel",)),
        in_specs=[row_spec, row_spec, col_spec, col_spec],
        out_specs=[col_spec, col_spec, row_spec],
        out_shape=[
            jax.ShapeDtypeStruct((b, 1), jnp.int32),
            jax.ShapeDtypeStruct((b, 1), jnp.float32),
            jax.ShapeDtypeStruct((b, v), jnp.int32),
        ],
    )(logits, feature_counts, fea_gates.reshape(b, 1), noise.reshape(b, 1))
    return word.reshape(b), prob.reshape(b), new_counts


# speedup vs baseline: 1.1503x; 1.0488x over previous
import functools

import jax
import jax.numpy as jnp
from jax import lax
from jax.experimental import pallas as pl
from jax.experimental.pallas import tpu as pltpu

TOPK = 5
FEA_GATE_TH = 0.15
ROW_BLOCK = 8
NEG_INF = float("-inf")
BIGP = 2**30
LANE = 128
STEP = 2 * LANE


def _insert(ts, ps, c, q):
    for j in range(TOPK):
        cond = c > ts[j]
        nt = jnp.where(cond, c, ts[j])
        nc = jnp.where(cond, ts[j], c)
        np_ = jnp.where(cond, q, ps[j])
        nq = jnp.where(cond, ps[j], q)
        ts[j], ps[j], c, q = nt, np_, nc, nq
    return ts, ps


def _decode_body(logits_ref, counts_ref, gates_ref, noise_ref,
                 word_ref, prob_ref, out_counts_ref):
    rb, v = logits_ref.shape
    ns = 4
    unroll = 4
    step = ns * unroll * LANE
    nfull = (v - LANE) // step
    base_tail = nfull * step
    lane_iota = lax.broadcasted_iota(jnp.int32, (rb, LANE), 1)
    lane4 = lane_iota << 2

    def slice_pair(base):
        c = logits_ref[:, pl.ds(base, LANE)]
        q = counts_ref[:, pl.ds(base, LANE)] + (lane4 + (base << 2))
        return c, q

    init = []
    for _ in range(ns):
        init += [jnp.full((rb, LANE), NEG_INF, jnp.float32)
                 for _ in range(TOPK)]
        init += [jnp.full((rb, LANE), BIGP, jnp.int32) for _ in range(TOPK)]

    def body(s, carry):
        carry = list(carry)
        base = pl.multiple_of(s * step, step)
        for k in range(ns * unroll):
            o = 2 * TOPK * (k % ns)
            ts, ps = carry[o:o + TOPK], carry[o + TOPK:o + 2 * TOPK]
            c, q = slice_pair(base + k * LANE)
            ts, ps = _insert(ts, ps, c, q)
            carry[o:o + TOPK] = ts
            carry[o + TOPK:o + 2 * TOPK] = ps
        return tuple(carry)

    carry = list(lax.fori_loop(0, nfull, body, tuple(init)))
    nfull_tail = (v - base_tail) // LANE
    for k in range(nfull_tail):
        o = 2 * TOPK * (k % ns)
        ts, ps = carry[o:o + TOPK], carry[o + TOPK:o + 2 * TOPK]
        c, q = slice_pair(base_tail + k * LANE)
        ts, ps = _insert(ts, ps, c, q)
        carry[o:o + TOPK] = ts
        carry[o + TOPK:o + 2 * TOPK] = ps
    novl = LANE - (v - base_tail - nfull_tail * LANE)
    keep = lane_iota >= novl
    if novl < LANE:
        c = logits_ref[:, pl.ds(v - LANE, LANE)]
        q = counts_ref[:, pl.ds(v - LANE, LANE)] + (lane4 + ((v - LANE) << 2))
        c = jnp.where(keep, c, NEG_INF)
        q = jnp.where(keep, q, BIGP)
        o = 2 * TOPK * (ns - 1)
        ts, ps = carry[o:o + TOPK], carry[o + TOPK:o + 2 * TOPK]
        ts, ps = _insert(ts, ps, c, q)
        carry[o:o + TOPK] = ts
        carry[o + TOPK:o + 2 * TOPK] = ps

    t_all = jnp.concatenate(
        [t for k in range(ns) for t in carry[2 * TOPK * k:2 * TOPK * k + TOPK]],
        axis=-1)
    p_all = jnp.concatenate(
        [p for k in range(ns)
         for p in carry[2 * TOPK * k + TOPK:2 * TOPK * (k + 1)]], axis=-1)
    vals, pks = [], []
    for _ in range(TOPK):
        vi = jnp.max(t_all, axis=-1, keepdims=True)
        pi = jnp.min(jnp.where(t_all == vi, p_all, BIGP), axis=-1,
                     keepdims=True)
        t_all = jnp.where((t_all == vi) & (p_all == pi), NEG_INF, t_all)
        vals.append(vi)
        pks.append(pi)
    idxs = [p >> 2 for p in pks]
    cnts = [p & 3 for p in pks]

    m0 = vals[0]
    s_sum = jnp.sum(jnp.exp(logits_ref[...] - m0), axis=-1, keepdims=True)

    qs = [jnp.exp(vi - m0) / s_sum for vi in vals]

    gate = gates_ref[...] > FEA_GATE_TH
    pprobs = [jnp.where(gate, qi / (1.0 + 2.0 * ci.astype(jnp.float32)), qi)
              for qi, ci in zip(qs, cnts)]

    total = pprobs[0] + pprobs[1] + pprobs[2] + pprobs[3] + pprobs[4]
    u = noise_ref[...] * total
    cdf = pprobs[0]
    choice = (cdf < u).astype(jnp.int32)
    for i in range(1, TOPK):
        cdf = cdf + pprobs[i]
        choice = choice + (cdf < u).astype(jnp.int32)
    choice = jnp.clip(choice, 0, TOPK - 1)

    word = idxs[TOPK - 1]
    prob = qs[TOPK - 1]
    for i in range(TOPK - 2, -1, -1):
        pick = choice == i
        word = jnp.where(pick, idxs[i], word)
        prob = jnp.where(pick, qs[i], prob)

    word_ref[...] = word
    prob_ref[...] = prob

    ckey = jnp.where(gate, word, -1)
    full_iota = lax.broadcasted_iota(jnp.int32, (rb, v), 1)
    out_counts_ref[...] = (counts_ref[...]
                           + (full_iota == ckey).astype(jnp.int32))


@functools.partial(jax.jit, static_argnames=())
def kernel(logits, feature_counts, fea_gates, noise):
    b, v = logits.shape
    rb = ROW_BLOCK
    grid = (b // rb,)
    row_spec = pl.BlockSpec((rb, v), lambda i: (i, 0))
    col_spec = pl.BlockSpec((rb, 1), lambda i: (i, 0))

    word, prob, new_counts = pl.pallas_call(
        _decode_body,
        grid=grid,
        compiler_params=pltpu.CompilerParams(
            dimension_semantics=("parallel",)),
        in_specs=[row_spec, row_spec, col_spec, col_spec],
        out_specs=[col_spec, col_spec, row_spec],
        out_shape=[
            jax.ShapeDtypeStruct((b, 1), jnp.int32),
            jax.ShapeDtypeStruct((b, 1), jnp.float32),
            jax.ShapeDtypeStruct((b, v), jnp.int32),
        ],
    )(logits, feature_counts, fea_gates.reshape(b, 1), noise.reshape(b, 1))
    return word.reshape(b), prob.reshape(b), new_counts


# speedup vs baseline: 1.1661x; 1.0138x over previous
import functools

import jax
import jax.numpy as jnp
from jax import lax
from jax.experimental import pallas as pl
from jax.experimental.pallas import tpu as pltpu

TOPK = 5
FEA_GATE_TH = 0.15
ROW_BLOCK = 8
NEG_INF = float("-inf")
BIGP = 2**30
LANE = 128
STEP = 2 * LANE


def _insert(ts, ps, c, q):
    for j in range(TOPK):
        cond = c > ts[j]
        nt = jnp.where(cond, c, ts[j])
        nc = jnp.where(cond, ts[j], c)
        np_ = jnp.where(cond, q, ps[j])
        nq = jnp.where(cond, ps[j], q)
        ts[j], ps[j], c, q = nt, np_, nc, nq
    return ts, ps


def _decode_body(logits_ref, counts_ref, gates_ref, noise_ref,
                 word_ref, prob_ref, out_counts_ref):
    rb, v = logits_ref.shape
    ns = 4
    unroll = 8
    step = ns * unroll * LANE
    nfull = (v - LANE) // step
    base_tail = nfull * step
    lane_iota = lax.broadcasted_iota(jnp.int32, (rb, LANE), 1)
    lane4 = lane_iota << 2

    def slice_pair(base):
        c = logits_ref[:, pl.ds(base, LANE)]
        q = counts_ref[:, pl.ds(base, LANE)] + (lane4 + (base << 2))
        return c, q

    init = []
    for _ in range(ns):
        init += [jnp.full((rb, LANE), NEG_INF, jnp.float32)
                 for _ in range(TOPK)]
        init += [jnp.full((rb, LANE), BIGP, jnp.int32) for _ in range(TOPK)]

    def body(s, carry):
        carry = list(carry)
        base = pl.multiple_of(s * step, step)
        for k in range(ns * unroll):
            o = 2 * TOPK * (k % ns)
            ts, ps = carry[o:o + TOPK], carry[o + TOPK:o + 2 * TOPK]
            c, q = slice_pair(base + k * LANE)
            ts, ps = _insert(ts, ps, c, q)
            carry[o:o + TOPK] = ts
            carry[o + TOPK:o + 2 * TOPK] = ps
        return tuple(carry)

    carry = list(lax.fori_loop(0, nfull, body, tuple(init)))
    nfull_tail = (v - base_tail) // LANE
    for k in range(nfull_tail):
        o = 2 * TOPK * (k % ns)
        ts, ps = carry[o:o + TOPK], carry[o + TOPK:o + 2 * TOPK]
        c, q = slice_pair(base_tail + k * LANE)
        ts, ps = _insert(ts, ps, c, q)
        carry[o:o + TOPK] = ts
        carry[o + TOPK:o + 2 * TOPK] = ps
    novl = LANE - (v - base_tail - nfull_tail * LANE)
    keep = lane_iota >= novl
    if novl < LANE:
        c = logits_ref[:, pl.ds(v - LANE, LANE)]
        q = counts_ref[:, pl.ds(v - LANE, LANE)] + (lane4 + ((v - LANE) << 2))
        c = jnp.where(keep, c, NEG_INF)
        q = jnp.where(keep, q, BIGP)
        o = 2 * TOPK * (ns - 1)
        ts, ps = carry[o:o + TOPK], carry[o + TOPK:o + 2 * TOPK]
        ts, ps = _insert(ts, ps, c, q)
        carry[o:o + TOPK] = ts
        carry[o + TOPK:o + 2 * TOPK] = ps

    t_all = jnp.concatenate(
        [t for k in range(ns) for t in carry[2 * TOPK * k:2 * TOPK * k + TOPK]],
        axis=-1)
    p_all = jnp.concatenate(
        [p for k in range(ns)
         for p in carry[2 * TOPK * k + TOPK:2 * TOPK * (k + 1)]], axis=-1)
    vals, pks = [], []
    for _ in range(TOPK):
        vi = jnp.max(t_all, axis=-1, keepdims=True)
        pi = jnp.min(jnp.where(t_all == vi, p_all, BIGP), axis=-1,
                     keepdims=True)
        t_all = jnp.where((t_all == vi) & (p_all == pi), NEG_INF, t_all)
        vals.append(vi)
        pks.append(pi)
    idxs = [p >> 2 for p in pks]
    cnts = [p & 3 for p in pks]

    m0 = vals[0]
    s_sum = jnp.sum(jnp.exp(logits_ref[...] - m0), axis=-1, keepdims=True)

    qs = [jnp.exp(vi - m0) / s_sum for vi in vals]

    gate = gates_ref[...] > FEA_GATE_TH
    pprobs = [jnp.where(gate, qi / (1.0 + 2.0 * ci.astype(jnp.float32)), qi)
              for qi, ci in zip(qs, cnts)]

    total = pprobs[0] + pprobs[1] + pprobs[2] + pprobs[3] + pprobs[4]
    u = noise_ref[...] * total
    cdf = pprobs[0]
    choice = (cdf < u).astype(jnp.int32)
    for i in range(1, TOPK):
        cdf = cdf + pprobs[i]
        choice = choice + (cdf < u).astype(jnp.int32)
    choice = jnp.clip(choice, 0, TOPK - 1)

    word = idxs[TOPK - 1]
    prob = qs[TOPK - 1]
    for i in range(TOPK - 2, -1, -1):
        pick = choice == i
        word = jnp.where(pick, idxs[i], word)
        prob = jnp.where(pick, qs[i], prob)

    word_ref[...] = word
    prob_ref[...] = prob

    ckey = jnp.where(gate, word, -1)
    full_iota = lax.broadcasted_iota(jnp.int32, (rb, v), 1)
    out_counts_ref[...] = (counts_ref[...]
                           + (full_iota == ckey).astype(jnp.int32))


@functools.partial(jax.jit, static_argnames=())
def kernel(logits, feature_counts, fea_gates, noise):
    b, v = logits.shape
    rb = ROW_BLOCK
    grid = (b // rb,)
    row_spec = pl.BlockSpec((rb, v), lambda i: (i, 0))
    col_spec = pl.BlockSpec((rb, 1), lambda i: (i, 0))

    word, prob, new_counts = pl.pallas_call(
        _decode_body,
        grid=grid,
        compiler_params=pltpu.CompilerParams(
            dimension_semantics=("parallel",)),
        in_specs=[row_spec, row_spec, col_spec, col_spec],
        out_specs=[col_spec, col_spec, row_spec],
        out_shape=[
            jax.ShapeDtypeStruct((b, 1), jnp.int32),
            jax.ShapeDtypeStruct((b, 1), jnp.float32),
            jax.ShapeDtypeStruct((b, v), jnp.int32),
        ],
    )(logits, feature_counts, fea_gates.reshape(b, 1), noise.reshape(b, 1))
    return word.reshape(b), prob.reshape(b), new_counts


# speedup vs baseline: 1.1742x; 1.0070x over previous
import functools

import jax
import jax.numpy as jnp
from jax import lax
from jax.experimental import pallas as pl
from jax.experimental.pallas import tpu as pltpu

TOPK = 5
FEA_GATE_TH = 0.15
ROW_BLOCK = 8
NEG_INF = float("-inf")
BIGP = 2**30
LANE = 128
STEP = 2 * LANE


def _insert(ts, ps, c, q):
    for j in range(TOPK):
        cond = c > ts[j]
        nt = jnp.where(cond, c, ts[j])
        nc = jnp.where(cond, ts[j], c)
        np_ = jnp.where(cond, q, ps[j])
        nq = jnp.where(cond, ps[j], q)
        ts[j], ps[j], c, q = nt, np_, nc, nq
    return ts, ps


def _decode_body(logits_ref, counts_ref, gates_ref, noise_ref,
                 word_ref, prob_ref, out_counts_ref):
    rb, v = logits_ref.shape
    ns = 4
    unroll = 16
    step = ns * unroll * LANE
    nfull = (v - LANE) // step
    base_tail = nfull * step
    lane_iota = lax.broadcasted_iota(jnp.int32, (rb, LANE), 1)
    lane4 = lane_iota << 2

    def slice_pair(base):
        c = logits_ref[:, pl.ds(base, LANE)]
        q = counts_ref[:, pl.ds(base, LANE)] + (lane4 + (base << 2))
        return c, q

    init = []
    for _ in range(ns):
        init += [jnp.full((rb, LANE), NEG_INF, jnp.float32)
                 for _ in range(TOPK)]
        init += [jnp.full((rb, LANE), BIGP, jnp.int32) for _ in range(TOPK)]

    def body(s, carry):
        carry = list(carry)
        base = pl.multiple_of(s * step, step)
        for k in range(ns * unroll):
            o = 2 * TOPK * (k % ns)
            ts, ps = carry[o:o + TOPK], carry[o + TOPK:o + 2 * TOPK]
            c, q = slice_pair(base + k * LANE)
            ts, ps = _insert(ts, ps, c, q)
            carry[o:o + TOPK] = ts
            carry[o + TOPK:o + 2 * TOPK] = ps
        return tuple(carry)

    carry = list(lax.fori_loop(0, nfull, body, tuple(init)))
    nfull_tail = (v - base_tail) // LANE
    for k in range(nfull_tail):
        o = 2 * TOPK * (k % ns)
        ts, ps = carry[o:o + TOPK], carry[o + TOPK:o + 2 * TOPK]
        c, q = slice_pair(base_tail + k * LANE)
        ts, ps = _insert(ts, ps, c, q)
        carry[o:o + TOPK] = ts
        carry[o + TOPK:o + 2 * TOPK] = ps
    novl = LANE - (v - base_tail - nfull_tail * LANE)
    keep = lane_iota >= novl
    if novl < LANE:
        c = logits_ref[:, pl.ds(v - LANE, LANE)]
        q = counts_ref[:, pl.ds(v - LANE, LANE)] + (lane4 + ((v - LANE) << 2))
        c = jnp.where(keep, c, NEG_INF)
        q = jnp.where(keep, q, BIGP)
        o = 2 * TOPK * (ns - 1)
        ts, ps = carry[o:o + TOPK], carry[o + TOPK:o + 2 * TOPK]
        ts, ps = _insert(ts, ps, c, q)
        carry[o:o + TOPK] = ts
        carry[o + TOPK:o + 2 * TOPK] = ps

    t_all = jnp.concatenate(
        [t for k in range(ns) for t in carry[2 * TOPK * k:2 * TOPK * k + TOPK]],
        axis=-1)
    p_all = jnp.concatenate(
        [p for k in range(ns)
         for p in carry[2 * TOPK * k + TOPK:2 * TOPK * (k + 1)]], axis=-1)
    vals, pks = [], []
    for _ in range(TOPK):
        vi = jnp.max(t_all, axis=-1, keepdims=True)
        pi = jnp.min(jnp.where(t_all == vi, p_all, BIGP), axis=-1,
                     keepdims=True)
        t_all = jnp.where((t_all == vi) & (p_all == pi), NEG_INF, t_all)
        vals.append(vi)
        pks.append(pi)
    idxs = [p >> 2 for p in pks]
    cnts = [p & 3 for p in pks]

    m0 = vals[0]
    s_sum = jnp.sum(jnp.exp(logits_ref[...] - m0), axis=-1, keepdims=True)

    qs = [jnp.exp(vi - m0) / s_sum for vi in vals]

    gate = gates_ref[...] > FEA_GATE_TH
    pprobs = [jnp.where(gate, qi / (1.0 + 2.0 * ci.astype(jnp.float32)), qi)
              for qi, ci in zip(qs, cnts)]

    total = pprobs[0] + pprobs[1] + pprobs[2] + pprobs[3] + pprobs[4]
    u = noise_ref[...] * total
    cdf = pprobs[0]
    choice = (cdf < u).astype(jnp.int32)
    for i in range(1, TOPK):
        cdf = cdf + pprobs[i]
        choice = choice + (cdf < u).astype(jnp.int32)
    choice = jnp.clip(choice, 0, TOPK - 1)

    word = idxs[TOPK - 1]
    prob = qs[TOPK - 1]
    for i in range(TOPK - 2, -1, -1):
        pick = choice == i
        word = jnp.where(pick, idxs[i], word)
        prob = jnp.where(pick, qs[i], prob)

    word_ref[...] = word
    prob_ref[...] = prob

    ckey = jnp.where(gate, word, -1)
    full_iota = lax.broadcasted_iota(jnp.int32, (rb, v), 1)
    out_counts_ref[...] = (counts_ref[...]
                           + (full_iota == ckey).astype(jnp.int32))


@functools.partial(jax.jit, static_argnames=())
def kernel(logits, feature_counts, fea_gates, noise):
    b, v = logits.shape
    rb = ROW_BLOCK
    grid = (b // rb,)
    row_spec = pl.BlockSpec((rb, v), lambda i: (i, 0))
    col_spec = pl.BlockSpec((rb, 1), lambda i: (i, 0))

    word, prob, new_counts = pl.pallas_call(
        _decode_body,
        grid=grid,
        compiler_params=pltpu.CompilerParams(
            dimension_semantics=("parallel",)),
        in_specs=[row_spec, row_spec, col_spec, col_spec],
        out_specs=[col_spec, col_spec, row_spec],
        out_shape=[
            jax.ShapeDtypeStruct((b, 1), jnp.int32),
            jax.ShapeDtypeStruct((b, 1), jnp.float32),
            jax.ShapeDtypeStruct((b, v), jnp.int32),
        ],
    )(logits, feature_counts, fea_gates.reshape(b, 1), noise.reshape(b, 1))
    return word.reshape(b), prob.reshape(b), new_counts


# speedup vs baseline: 1.1749x; 1.0005x over previous
import functools

import jax
import jax.numpy as jnp
from jax import lax
from jax.experimental import pallas as pl
from jax.experimental.pallas import tpu as pltpu

TOPK = 5
FEA_GATE_TH = 0.15
ROW_BLOCK = 8
NEG_INF = float("-inf")
BIGP = 2**30
LANE = 128
STEP = 2 * LANE


ESHIFT = 12.0


def _insert(ts, ps, c, q):
    for j in range(TOPK - 1):
        cond = c > ts[j]
        nt = jnp.where(cond, c, ts[j])
        nc = jnp.where(cond, ts[j], c)
        np_ = jnp.where(cond, q, ps[j])
        nq = jnp.where(cond, ps[j], q)
        ts[j], ps[j], c, q = nt, np_, nc, nq
    cond = c > ts[TOPK - 1]
    ts[TOPK - 1] = jnp.where(cond, c, ts[TOPK - 1])
    ps[TOPK - 1] = jnp.where(cond, q, ps[TOPK - 1])
    return ts, ps


def _decode_body(logits_ref, counts_ref, gates_ref, noise_ref,
                 word_ref, prob_ref, out_counts_ref):
    rb, v = logits_ref.shape
    ns = 4
    unroll = 16
    step = ns * unroll * LANE
    nfull = (v - LANE) // step
    base_tail = nfull * step
    lane_iota = lax.broadcasted_iota(jnp.int32, (rb, LANE), 1)
    lane4 = lane_iota << 2

    def slice_pair(base):
        c = logits_ref[:, pl.ds(base, LANE)]
        q = counts_ref[:, pl.ds(base, LANE)] + (lane4 + (base << 2))
        return c, q

    init = []
    for _ in range(ns):
        init += [jnp.full((rb, LANE), NEG_INF, jnp.float32)
                 for _ in range(TOPK)]
        init += [jnp.full((rb, LANE), BIGP, jnp.int32) for _ in range(TOPK)]
    init += [jnp.zeros((rb, LANE), jnp.float32) for _ in range(ns)]

    def body(s, carry):
        carry = list(carry)
        base = pl.multiple_of(s * step, step)
        accs = list(carry[2 * TOPK * ns:])
        for k in range(ns * unroll):
            o = 2 * TOPK * (k % ns)
            ts, ps = carry[o:o + TOPK], carry[o + TOPK:o + 2 * TOPK]
            c, q = slice_pair(base + k * LANE)
            ts, ps = _insert(ts, ps, c, q)
            accs[k % ns] = accs[k % ns] + jnp.exp(c - ESHIFT)
            carry[o:o + TOPK] = ts
            carry[o + TOPK:o + 2 * TOPK] = ps
        carry[2 * TOPK * ns:] = accs
        return tuple(carry)

    carry = list(lax.fori_loop(0, nfull, body, tuple(init)))
    nfull_tail = (v - base_tail) // LANE
    accs = list(carry[2 * TOPK * ns:])
    for k in range(nfull_tail):
        o = 2 * TOPK * (k % ns)
        ts, ps = carry[o:o + TOPK], carry[o + TOPK:o + 2 * TOPK]
        c, q = slice_pair(base_tail + k * LANE)
        ts, ps = _insert(ts, ps, c, q)
        accs[k % ns] = accs[k % ns] + jnp.exp(c - ESHIFT)
        carry[o:o + TOPK] = ts
        carry[o + TOPK:o + 2 * TOPK] = ps
    novl = LANE - (v - base_tail - nfull_tail * LANE)
    keep = lane_iota >= novl
    if novl < LANE:
        c = logits_ref[:, pl.ds(v - LANE, LANE)]
        q = counts_ref[:, pl.ds(v - LANE, LANE)] + (lane4 + ((v - LANE) << 2))
        c = jnp.where(keep, c, NEG_INF)
        q = jnp.where(keep, q, BIGP)
        o = 2 * TOPK * (ns - 1)
        ts, ps = carry[o:o + TOPK], carry[o + TOPK:o + 2 * TOPK]
        ts, ps = _insert(ts, ps, c, q)
        accs[ns - 1] = accs[ns - 1] + jnp.exp(c - ESHIFT)
        carry[o:o + TOPK] = ts
        carry[o + TOPK:o + 2 * TOPK] = ps

    t_all = jnp.concatenate(
        [t for k in range(ns) for t in carry[2 * TOPK * k:2 * TOPK * k + TOPK]],
        axis=-1)
    p_all = jnp.concatenate(
        [p for k in range(ns)
         for p in carry[2 * TOPK * k + TOPK:2 * TOPK * (k + 1)]], axis=-1)
    vals, pks = [], []
    for _ in range(TOPK):
        vi = jnp.max(t_all, axis=-1, keepdims=True)
        pi = jnp.min(jnp.where(t_all == vi, p_all, BIGP), axis=-1,
                     keepdims=True)
        t_all = jnp.where((t_all == vi) & (p_all == pi), NEG_INF, t_all)
        vals.append(vi)
        pks.append(pi)
    idxs = [p >> 2 for p in pks]
    cnts = [p & 3 for p in pks]

    m0 = vals[0]
    acc_all = accs[0] + accs[1] + accs[2] + accs[3]
    s_sum = (jnp.sum(acc_all, axis=-1, keepdims=True)
             * jnp.exp(ESHIFT - m0))

    qs = [jnp.exp(vi - m0) / s_sum for vi in vals]

    gate = gates_ref[...] > FEA_GATE_TH
    pprobs = [jnp.where(gate, qi / (1.0 + 2.0 * ci.astype(jnp.float32)), qi)
              for qi, ci in zip(qs, cnts)]

    total = pprobs[0] + pprobs[1] + pprobs[2] + pprobs[3] + pprobs[4]
    u = noise_ref[...] * total
    cdf = pprobs[0]
    choice = (cdf < u).astype(jnp.int32)
    for i in range(1, TOPK):
        cdf = cdf + pprobs[i]
        choice = choice + (cdf < u).astype(jnp.int32)
    choice = jnp.clip(choice, 0, TOPK - 1)

    word = idxs[TOPK - 1]
    prob = qs[TOPK - 1]
    for i in range(TOPK - 2, -1, -1):
        pick = choice == i
        word = jnp.where(pick, idxs[i], word)
        prob = jnp.where(pick, qs[i], prob)

    word_ref[...] = word
    prob_ref[...] = prob

    ckey = jnp.where(gate, word, -1)
    full_iota = lax.broadcasted_iota(jnp.int32, (rb, v), 1)
    out_counts_ref[...] = (counts_ref[...]
                           + (full_iota == ckey).astype(jnp.int32))


@functools.partial(jax.jit, static_argnames=())
def kernel(logits, feature_counts, fea_gates, noise):
    b, v = logits.shape
    rb = ROW_BLOCK
    grid = (b // rb,)
    row_spec = pl.BlockSpec((rb, v), lambda i: (i, 0))
    col_spec = pl.BlockSpec((rb, 1), lambda i: (i, 0))

    word, prob, new_counts = pl.pallas_call(
        _decode_body,
        grid=grid,
        compiler_params=pltpu.CompilerParams(
            dimension_semantics=("parallel",)),
        in_specs=[row_spec, row_spec, col_spec, col_spec],
        out_specs=[col_spec, col_spec, row_spec],
        out_shape=[
            jax.ShapeDtypeStruct((b, 1), jnp.int32),
            jax.ShapeDtypeStruct((b, 1), jnp.float32),
            jax.ShapeDtypeStruct((b, v), jnp.int32),
        ],
    )(logits, feature_counts, fea_gates.reshape(b, 1), noise.reshape(b, 1))
    return word.reshape(b), prob.reshape(b), new_counts
